# baseline (speedup 1.0000x reference)
import functools

import jax
import jax.numpy as jnp
from jax import lax
from jax.experimental import pallas as pl
from jax.experimental.pallas import tpu as pltpu
from jax.experimental.pallas import tpu_sc as plsc

N = 10000
E = 320000
F = 128
C = 47

NC = 2
NS = 16
NW = NC * NS
K = 128
CB = 8
NB = 10
CH = CB * NB
EPT = CH * K
EPAD = NW * EPT
NPAD = 10112
RPT = NPAD // NS
ROW_OFFS = (0, 128, 256, 384, RPT - 128)


def _agg_body(with_deg, *refs):
    if with_deg:
        (table, src2d, dst2d, acc_out, deg_out, acc_sh, deg_sh,
         sidx, didx, rows_a, rows_b, ones1, zrow,
         sem_ga, sem_gb, sem_sa, sem_sb) = refs
    else:
        (table, src2d, dst2d, acc_out, acc_sh,
         sidx, didx, rows_a, rows_b,
         sem_ga, sem_gb, sem_sa, sem_sb) = refs

    c = lax.axis_index("c")
    s = lax.axis_index("s")
    w = c * NS + s

    @pl.loop(0, K)
    def _fill(i):
        for j in range(F // 16):
            rows_a[i, pl.ds(j * 16, 16)] = jnp.zeros((16,), jnp.float32)

    if with_deg:
        @pl.loop(0, K // 16)
        def _fill_ones(i):
            ones1[pl.ds(i * 16, 16)] = jnp.ones((16,), jnp.float32)

        @pl.loop(0, RPT // 16)
        def _fill_z(i):
            zrow[pl.ds(i * 16, 16)] = jnp.zeros((16,), jnp.float32)

        zrow[pl.ds(RPT - 16, 16)] = jnp.zeros((16,), jnp.float32)

    zbase = s * RPT
    for off in ROW_OFFS:
        pltpu.sync_copy(rows_a, acc_sh.at[pl.ds(zbase + off, 128)])
    if with_deg:
        pltpu.sync_copy(zrow, deg_sh.at[pl.ds(zbase, RPT)])
    plsc.subcore_barrier()

    def stage(b):
        ibase = w * CH + b * CB
        pltpu.sync_copy(src2d.at[pl.ds(ibase, CB)], sidx)
        pltpu.sync_copy(dst2d.at[pl.ds(ibase, CB)], didx.at[b % 2])

    def fire_g(j, rows, sem):
        pltpu.async_copy(table.at[sidx.at[j]], rows, sem)

    def drain_g(rows, sem):
        pltpu.make_async_copy(table.at[sidx.at[0]], rows, sem).wait()

    def fire_s(slot, j, rows, sem):
        pltpu.async_copy(rows, acc_sh.at[didx.at[slot, j]], sem, add=True)

    def drain_s(rows, sem):
        pltpu.make_async_copy(table.at[sidx.at[0]], rows, sem).wait()

    def deg_add(slot, j):
        if with_deg:
            pltpu.sync_copy(ones1, deg_sh.at[didx.at[slot, j]], add=True)

    stage(0)
    fire_g(0, rows_a, sem_ga)
    fire_g(1, rows_b, sem_gb)

    @pl.loop(0, NB)
    def _blk(b):
        slot = b % 2

        @pl.loop(0, CB // 2 - 1)
        def _pair(p):
            drain_g(rows_a, sem_ga)
            fire_s(slot, 2 * p, rows_a, sem_sa)
            deg_add(slot, 2 * p)
            drain_g(rows_b, sem_gb)
            fire_s(slot, 2 * p + 1, rows_b, sem_sb)
            deg_add(slot, 2 * p + 1)
            drain_s(rows_a, sem_sa)
            fire_g(2 * p + 2, rows_a, sem_ga)
            drain_s(rows_b, sem_sb)
            fire_g(2 * p + 3, rows_b, sem_gb)

        drain_g(rows_a, sem_ga)
        fire_s(slot, CB - 2, rows_a, sem_sa)
        deg_add(slot, CB - 2)
        drain_g(rows_b, sem_gb)
        fire_s(slot, CB - 1, rows_b, sem_sb)
        deg_add(slot, CB - 1)

        @pl.when(b < NB - 1)
        def _():
            stage(b + 1)
            drain_s(rows_a, sem_sa)
            fire_g(0, rows_a, sem_ga)
            drain_s(rows_b, sem_sb)
            fire_g(1, rows_b, sem_gb)

        @pl.when(b == NB - 1)
        def _():
            drain_s(rows_a, sem_sa)
            drain_s(rows_b, sem_sb)

    plsc.subcore_barrier()

    obase = c * NPAD + zbase
    for off in ROW_OFFS:
        pltpu.sync_copy(acc_sh.at[pl.ds(zbase + off, 128)], rows_a)
        pltpu.sync_copy(rows_a, acc_out.at[pl.ds(obase + off, 128)])
    if with_deg:
        pltpu.sync_copy(deg_sh.at[pl.ds(zbase, RPT)], zrow)
        pltpu.sync_copy(zrow, deg_out.at[pl.ds(c * NPAD + zbase, RPT)])


def _make_agg(with_deg):
    out_type = [jax.ShapeDtypeStruct((NC * NPAD, F), jnp.float32)]
    scratch = [pltpu.VMEM_SHARED((NPAD, F), jnp.float32)]
    if with_deg:
        out_type.append(jax.ShapeDtypeStruct((NC * NPAD,), jnp.float32))
        scratch.append(pltpu.VMEM_SHARED((NPAD,), jnp.float32))
    scratch += [
        pltpu.VMEM((CB, K), jnp.int32),
        pltpu.VMEM((2, CB, K), jnp.int32),
        pltpu.VMEM((K, F), jnp.float32),
        pltpu.VMEM((K, F), jnp.float32),
    ]
    if with_deg:
        scratch += [
            pltpu.VMEM((K,), jnp.float32),
            pltpu.VMEM((RPT,), jnp.float32),
        ]
    scratch += [pltpu.SemaphoreType.DMA] * 4
    return pl.kernel(
        functools.partial(_agg_body, with_deg),
        out_type=out_type,
        mesh=plsc.VectorSubcoreMesh(core_axis_name="c", subcore_axis_name="s"),
        scratch_types=scratch,
    )


_agg_deg = _make_agg(True)
_agg = _make_agg(False)


def _tc_layer_body(relu, h, aggA, aggB, degA, degB, wsT, wnT, o):
    deg = degA[:, 0:1] + degB[:, 0:1]
    inv = 1.0 / jnp.maximum(deg, 1.0)
    hn = (aggA[...] + aggB[...]) * inv
    acc = (jnp.dot(h[...], wsT[...], preferred_element_type=jnp.float32)
           + jnp.dot(hn, wnT[...], preferred_element_type=jnp.float32))
    if relu:
        acc = jnp.maximum(acc, 0.0)
    o[...] = acc


def _make_tc_layer(relu, out_w):
    R = 1000
    return pl.pallas_call(
        functools.partial(_tc_layer_body, relu),
        grid=(N // R,),
        in_specs=[
            pl.BlockSpec((R, F), lambda i: (i, 0)),
            pl.BlockSpec((R, F), lambda i: (i, 0)),
            pl.BlockSpec((R, F), lambda i: (i, 0)),
            pl.BlockSpec((R, 16), lambda i: (i, 0)),
            pl.BlockSpec((R, 16), lambda i: (i, 0)),
            pl.BlockSpec((F, out_w), lambda i: (0, 0)),
            pl.BlockSpec((F, out_w), lambda i: (0, 0)),
        ],
        out_specs=pl.BlockSpec((R, out_w), lambda i: (i, 0)),
        out_shape=jax.ShapeDtypeStruct((N, out_w), jnp.float32),
    )


_tc_layer0 = _make_tc_layer(True, F)
_tc_layer1 = _make_tc_layer(False, F)


def kernel(x, edge_index, W_self0, W_neigh0, W_self1, W_neigh1):
    src = edge_index[0].astype(jnp.int32)
    dst = edge_index[1].astype(jnp.int32)
    npad = EPAD - E
    pad_iota = jnp.arange(npad, dtype=jnp.int32)
    src2d = jnp.concatenate([src, pad_iota % N]).reshape(EPAD // K, K)
    dst2d = jnp.concatenate(
        [dst, N + pad_iota % (NPAD - N)]).reshape(EPAD // K, K)

    acc0, deg = _agg_deg(x, src2d, dst2d)
    deg16 = jnp.broadcast_to(deg[:, None], (NC * NPAD, 16))
    degA, degB = deg16[:NPAD], deg16[NPAD:]
    h1 = _tc_layer0(x, acc0[:NPAD], acc0[NPAD:], degA, degB,
                    W_self0.T, W_neigh0.T)

    (acc1,) = _agg(h1, src2d, dst2d)
    ws1 = jnp.zeros((F, F), jnp.float32).at[:, :C].set(W_self1.T)
    wn1 = jnp.zeros((F, F), jnp.float32).at[:, :C].set(W_neigh1.T)
    out = _tc_layer1(h1, acc1[:NPAD], acc1[NPAD:], degA, degB, ws1, wn1)
    return out[:, :C]


# speedup vs baseline: 1.0643x; 1.0643x over previous
import functools

import jax
import jax.numpy as jnp
from jax import lax
from jax.experimental import pallas as pl
from jax.experimental.pallas import tpu as pltpu
from jax.experimental.pallas import tpu_sc as plsc

N = 10000
E = 320000
F = 128
C = 47

NC = 2
NS = 16
NW = NC * NS
K = 128
CB = 16
NB = 5
CH = CB * NB
EPT = CH * K
EPAD = NW * EPT
NPAD = 10112
RPT = NPAD // NS
ROW_OFFS = (0, 128, 256, 384, RPT - 128)


def _agg_body(table, src2d, dst2d, acc_out,
              acc_sh, sidx, didx, rows_a, rows_b, sem_a, sem_b):
    c = lax.axis_index("c")
    s = lax.axis_index("s")
    w = c * NS + s

    @pl.loop(0, K)
    def _fill(i):
        for j in range(F // 16):
            rows_a[i, pl.ds(j * 16, 16)] = jnp.zeros((16,), jnp.float32)

    zbase = s * RPT
    for off in ROW_OFFS:
        pltpu.sync_copy(rows_a, acc_sh.at[pl.ds(zbase + off, 128)])
    plsc.subcore_barrier()

    def stage(b):
        ibase = w * CH + b * CB
        pltpu.sync_copy(src2d.at[pl.ds(ibase, CB)], sidx)
        pltpu.sync_copy(dst2d.at[pl.ds(ibase, CB)], didx)

    def fire(j, rows, sem):
        pltpu.async_copy(table.at[sidx.at[j]], rows, sem)

    def drain(rows, sem):
        pltpu.make_async_copy(table.at[sidx.at[0]], rows, sem).wait()

    def scatter(j, rows):
        pltpu.sync_copy(rows, acc_sh.at[didx.at[j]], add=True)

    stage(0)
    fire(0, rows_a, sem_a)

    @pl.loop(0, NB)
    def _blk(b):
        @pl.loop(0, CB // 2 - 1)
        def _pair(p):
            drain(rows_a, sem_a)
            fire(2 * p + 1, rows_b, sem_b)
            scatter(2 * p, rows_a)
            drain(rows_b, sem_b)
            fire(2 * p + 2, rows_a, sem_a)
            scatter(2 * p + 1, rows_b)

        drain(rows_a, sem_a)
        fire(CB - 1, rows_b, sem_b)
        scatter(CB - 2, rows_a)
        drain(rows_b, sem_b)
        scatter(CB - 1, rows_b)

        @pl.when(b < NB - 1)
        def _():
            stage(b + 1)
            fire(0, rows_a, sem_a)

    plsc.subcore_barrier()

    obase = c * NPAD + zbase
    pltpu.sync_copy(acc_sh.at[pl.ds(zbase, RPT)],
                    acc_out.at[pl.ds(obase, RPT)])


_agg = pl.kernel(
    _agg_body,
    out_type=[jax.ShapeDtypeStruct((NC * NPAD, F), jnp.float32)],
    mesh=plsc.VectorSubcoreMesh(core_axis_name="c", subcore_axis_name="s"),
    scratch_types=[
        pltpu.VMEM_SHARED((NPAD, F), jnp.float32),
        pltpu.VMEM((CB, K), jnp.int32),
        pltpu.VMEM((CB, K), jnp.int32),
        pltpu.VMEM((K, F), jnp.float32),
        pltpu.VMEM((K, F), jnp.float32),
        pltpu.SemaphoreType.DMA,
        pltpu.SemaphoreType.DMA,
    ],
)


def _deg_body(dst2d, deg_out, deg_sh, didx, ones1, zrow):
    c = lax.axis_index("c")
    s = lax.axis_index("s")
    w = c * NS + s

    @pl.loop(0, K // 16)
    def _fill1(i):
        ones1[pl.ds(i * 16, 16)] = jnp.ones((16,), jnp.float32)

    @pl.loop(0, RPT // 16)
    def _fill2(i):
        zrow[pl.ds(i * 16, 16)] = jnp.zeros((16,), jnp.float32)

    zrow[pl.ds(RPT - 16, 16)] = jnp.zeros((16,), jnp.float32)

    zbase = s * RPT
    pltpu.sync_copy(zrow, deg_sh.at[pl.ds(zbase, RPT)])
    plsc.subcore_barrier()

    @pl.loop(0, NB)
    def _blk(b):
        pltpu.sync_copy(dst2d.at[pl.ds(w * CH + b * CB, CB)], didx)

        @pl.loop(0, CB)
        def _chunk(j):
            pltpu.sync_copy(ones1, deg_sh.at[didx.at[j]], add=True)

    plsc.subcore_barrier()

    pltpu.sync_copy(deg_sh.at[pl.ds(zbase, RPT)], zrow)
    pltpu.sync_copy(zrow, deg_out.at[pl.ds(c * NPAD + zbase, RPT)])


_deg = pl.kernel(
    _deg_body,
    out_type=[jax.ShapeDtypeStruct((NC * NPAD,), jnp.float32)],
    mesh=plsc.VectorSubcoreMesh(core_axis_name="c", subcore_axis_name="s"),
    scratch_types=[
        pltpu.VMEM_SHARED((NPAD,), jnp.float32),
        pltpu.VMEM((CB, K), jnp.int32),
        pltpu.VMEM((K,), jnp.float32),
        pltpu.VMEM((RPT,), jnp.float32),
    ],
)


def _tc_layer_body(relu, h, aggA, aggB, degA, degB, wsT, wnT, o):
    deg = degA[:, 0:1] + degB[:, 0:1]
    inv = 1.0 / jnp.maximum(deg, 1.0)
    hn = (aggA[...] + aggB[...]) * inv
    acc = (jnp.dot(h[...], wsT[...], preferred_element_type=jnp.float32)
           + jnp.dot(hn, wnT[...], preferred_element_type=jnp.float32))
    if relu:
        acc = jnp.maximum(acc, 0.0)
    o[...] = acc


def _make_tc_layer(relu, out_w):
    R = 1000
    return pl.pallas_call(
        functools.partial(_tc_layer_body, relu),
        grid=(N // R,),
        in_specs=[
            pl.BlockSpec((R, F), lambda i: (i, 0)),
            pl.BlockSpec((R, F), lambda i: (i, 0)),
            pl.BlockSpec((R, F), lambda i: (i, 0)),
            pl.BlockSpec((R, 16), lambda i: (i, 0)),
            pl.BlockSpec((R, 16), lambda i: (i, 0)),
            pl.BlockSpec((F, out_w), lambda i: (0, 0)),
            pl.BlockSpec((F, out_w), lambda i: (0, 0)),
        ],
        out_specs=pl.BlockSpec((R, out_w), lambda i: (i, 0)),
        out_shape=jax.ShapeDtypeStruct((N, out_w), jnp.float32),
    )


_tc_layer0 = _make_tc_layer(True, F)
_tc_layer1 = _make_tc_layer(False, F)


def kernel(x, edge_index, W_self0, W_neigh0, W_self1, W_neigh1):
    src = edge_index[0].astype(jnp.int32)
    dst = edge_index[1].astype(jnp.int32)
    npad = EPAD - E
    pad_iota = jnp.arange(npad, dtype=jnp.int32)
    src2d = jnp.concatenate([src, pad_iota % N]).reshape(EPAD // K, K)
    dst2d = jnp.concatenate(
        [dst, N + pad_iota % (NPAD - N)]).reshape(EPAD // K, K)

    (deg,) = _deg(dst2d)
    (acc0,) = _agg(x, src2d, dst2d)
    deg16 = jnp.broadcast_to(deg[:, None], (NC * NPAD, 16))
    degA, degB = deg16[:NPAD], deg16[NPAD:]
    h1 = _tc_layer0(x, acc0[:NPAD], acc0[NPAD:], degA, degB,
                    W_self0.T, W_neigh0.T)

    (acc1,) = _agg(h1, src2d, dst2d)
    ws1 = jnp.zeros((F, F), jnp.float32).at[:, :C].set(W_self1.T)
    wn1 = jnp.zeros((F, F), jnp.float32).at[:, :C].set(W_neigh1.T)
    out = _tc_layer1(h1, acc1[:NPAD], acc1[NPAD:], degA, degB, ws1, wn1)
    return out[:, :C]


# speedup vs baseline: 1.0785x; 1.0133x over previous
import functools

import jax
import jax.numpy as jnp
from jax import lax
from jax.experimental import pallas as pl
from jax.experimental.pallas import tpu as pltpu
from jax.experimental.pallas import tpu_sc as plsc

N = 10000
E = 320000
F = 128
C = 47

NC = 2
NS = 16
NW = NC * NS
K = 128
CB = 16
NB = 5
CH = CB * NB
EPT = CH * K
EPAD = NW * EPT
NPAD = 10112
RPT = NPAD // NS
ROW_OFFS = (0, 128, 256, 384, RPT - 128)


def _agg_body(table, src2d, dst2d, acc_out,
              acc_sh, sidx, didx, rows_a, rows_b, sem_a, sem_b):
    c = lax.axis_index("c")
    s = lax.axis_index("s")
    w = c * NS + s

    @pl.loop(0, K)
    def _fill(i):
        for j in range(F // 16):
            rows_a[i, pl.ds(j * 16, 16)] = jnp.zeros((16,), jnp.float32)

    zbase = s * RPT
    for off in ROW_OFFS:
        pltpu.sync_copy(rows_a, acc_sh.at[pl.ds(zbase + off, 128)])
    plsc.subcore_barrier()

    def stage(b):
        ibase = w * CH + b * CB
        pltpu.sync_copy(src2d.at[pl.ds(ibase, CB)], sidx)
        pltpu.sync_copy(dst2d.at[pl.ds(ibase, CB)], didx)

    def fire(j, rows, sem):
        pltpu.async_copy(table.at[sidx.at[j]], rows, sem)

    def drain(rows, sem):
        pltpu.make_async_copy(table.at[sidx.at[0]], rows, sem).wait()

    def scatter(j, rows):
        pltpu.sync_copy(rows, acc_sh.at[didx.at[j]], add=True)

    stage(0)
    fire(0, rows_a, sem_a)

    @pl.loop(0, NB)
    def _blk(b):
        @pl.loop(0, CB // 2 - 1)
        def _pair(p):
            drain(rows_a, sem_a)
            fire(2 * p + 1, rows_b, sem_b)
            scatter(2 * p, rows_a)
            drain(rows_b, sem_b)
            fire(2 * p + 2, rows_a, sem_a)
            scatter(2 * p + 1, rows_b)

        drain(rows_a, sem_a)
        fire(CB - 1, rows_b, sem_b)
        scatter(CB - 2, rows_a)
        drain(rows_b, sem_b)
        scatter(CB - 1, rows_b)

        @pl.when(b < NB - 1)
        def _():
            stage(b + 1)
            fire(0, rows_a, sem_a)

    plsc.subcore_barrier()

    obase = c * NPAD + zbase
    pltpu.sync_copy(acc_sh.at[pl.ds(zbase, RPT)],
                    acc_out.at[pl.ds(obase, RPT)])


_agg = pl.kernel(
    _agg_body,
    out_type=[jax.ShapeDtypeStruct((NC * NPAD, F), jnp.float32)],
    mesh=plsc.VectorSubcoreMesh(core_axis_name="c", subcore_axis_name="s"),
    scratch_types=[
        pltpu.VMEM_SHARED((NPAD, F), jnp.float32),
        pltpu.VMEM((CB, K), jnp.int32),
        pltpu.VMEM((CB, K), jnp.int32),
        pltpu.VMEM((K, F), jnp.float32),
        pltpu.VMEM((K, F), jnp.float32),
        pltpu.SemaphoreType.DMA,
        pltpu.SemaphoreType.DMA,
    ],
)


def _deg_body(dst2d, deg_out, deg_sh, didx, ones1, zrow):
    c = lax.axis_index("c")
    s = lax.axis_index("s")
    w = c * NS + s

    @pl.loop(0, K // 16)
    def _fill1(i):
        ones1[pl.ds(i * 16, 16)] = jnp.ones((16,), jnp.float32)

    @pl.loop(0, RPT // 16)
    def _fill2(i):
        zrow[pl.ds(i * 16, 16)] = jnp.zeros((16,), jnp.float32)

    zrow[pl.ds(RPT - 16, 16)] = jnp.zeros((16,), jnp.float32)

    zbase = s * RPT
    pltpu.sync_copy(zrow, deg_sh.at[pl.ds(zbase, RPT)])
    plsc.subcore_barrier()

    @pl.loop(0, NB)
    def _blk(b):
        pltpu.sync_copy(dst2d.at[pl.ds(w * CH + b * CB, CB)], didx)

        @pl.loop(0, CB)
        def _chunk(j):
            pltpu.sync_copy(ones1, deg_sh.at[didx.at[j]], add=True)

    plsc.subcore_barrier()

    pltpu.sync_copy(deg_sh.at[pl.ds(zbase, RPT)], zrow)
    pltpu.sync_copy(zrow, deg_out.at[pl.ds(c * NPAD + zbase, RPT)])


_deg = pl.kernel(
    _deg_body,
    out_type=[jax.ShapeDtypeStruct((NC * NPAD,), jnp.float32)],
    mesh=plsc.VectorSubcoreMesh(core_axis_name="c", subcore_axis_name="s"),
    scratch_types=[
        pltpu.VMEM_SHARED((NPAD,), jnp.float32),
        pltpu.VMEM((CB, K), jnp.int32),
        pltpu.VMEM((K,), jnp.float32),
        pltpu.VMEM((RPT,), jnp.float32),
    ],
)


def _tc_layer_body(relu, h, aggA, aggB, degA, degB, wsT, wnT, o):
    deg = degA[:, 0:1] + degB[:, 0:1]
    inv = 1.0 / jnp.maximum(deg, 1.0)
    hn = (aggA[...] + aggB[...]) * inv
    acc = (jnp.dot(h[...], wsT[...], preferred_element_type=jnp.float32)
           + jnp.dot(hn, wnT[...], preferred_element_type=jnp.float32))
    if relu:
        acc = jnp.maximum(acc, 0.0)
    o[...] = acc


def _make_tc_layer(relu, out_w):
    R = RPT
    return pl.pallas_call(
        functools.partial(_tc_layer_body, relu),
        grid=(NPAD // R,),
        in_specs=[
            pl.BlockSpec((R, F), lambda i: (i, 0)),
            pl.BlockSpec((R, F), lambda i: (i, 0)),
            pl.BlockSpec((R, F), lambda i: (i + NS, 0)),
            pl.BlockSpec((R, 16), lambda i: (i, 0)),
            pl.BlockSpec((R, 16), lambda i: (i + NS, 0)),
            pl.BlockSpec((F, out_w), lambda i: (0, 0)),
            pl.BlockSpec((F, out_w), lambda i: (0, 0)),
        ],
        out_specs=pl.BlockSpec((R, out_w), lambda i: (i, 0)),
        out_shape=jax.ShapeDtypeStruct((N, out_w), jnp.float32),
    )


_tc_layer0 = _make_tc_layer(True, F)
_tc_layer1 = _make_tc_layer(False, F)


def kernel(x, edge_index, W_self0, W_neigh0, W_self1, W_neigh1):
    src = edge_index[0].astype(jnp.int32)
    dst = edge_index[1].astype(jnp.int32)
    npad = EPAD - E
    pad_iota = jnp.arange(npad, dtype=jnp.int32)
    src2d = jnp.concatenate([src, pad_iota % N]).reshape(EPAD // K, K)
    dst2d = jnp.concatenate(
        [dst, N + pad_iota % (NPAD - N)]).reshape(EPAD // K, K)

    (deg,) = _deg(dst2d)
    (acc0,) = _agg(x, src2d, dst2d)
    deg16 = jnp.broadcast_to(deg[:, None], (NC * NPAD, 16))
    h1 = _tc_layer0(x, acc0, acc0, deg16, deg16, W_self0.T, W_neigh0.T)

    (acc1,) = _agg(h1, src2d, dst2d)
    ws1 = jnp.zeros((F, F), jnp.float32).at[:, :C].set(W_self1.T)
    wn1 = jnp.zeros((F, F), jnp.float32).at[:, :C].set(W_neigh1.T)
    out = _tc_layer1(h1, acc1, acc1, deg16, deg16, ws1, wn1)
    return out[:, :C]
